# Initial kernel scaffold; baseline (speedup 1.0000x reference)
import jax
import jax.numpy as jnp
from jax.experimental import pallas as pl


def kernel(x, conv1_w, bn1_g, bn1_b, conv2_w, bn2_g, bn2_b, fc1_w, fc1_b, fc2_w, fc2_b):
    raise NotImplementedError("write your pallas kernel here")


# speedup vs baseline: 1.7622x; 1.7622x over previous
import functools

import jax
import jax.numpy as jnp
from jax import lax
from jax.experimental import pallas as pl
from jax.experimental.pallas import tpu as pltpu

_VMEM_LIMIT = 48 * 1024 * 1024


def _conv_body(xb, w_ref, W, P, C):
    z1 = jnp.zeros((C, 1), xb.dtype)
    ww = lax.broadcasted_iota(jnp.int32, (1, P), 1) % W
    left = jnp.concatenate([xb[:, 1:], z1], axis=1)
    right = jnp.concatenate([z1, xb[:, :-1]], axis=1)
    left = jnp.where(ww < W - 1, left, jnp.zeros_like(left))
    right = jnp.where(ww >= 1, right, jnp.zeros_like(right))
    x3 = jnp.concatenate([right, xb, left], axis=0)

    b0 = jnp.dot(w_ref[:, 0:3 * C], x3, preferred_element_type=jnp.float32)
    b1 = jnp.dot(w_ref[:, 3 * C:6 * C], x3, preferred_element_type=jnp.float32)
    b2 = jnp.dot(w_ref[:, 6 * C:9 * C], x3, preferred_element_type=jnp.float32)
    zw = jnp.zeros((C, W), jnp.float32)
    acc = (jnp.concatenate([zw, b0[:, :P - W]], axis=1) + b1
           + jnp.concatenate([b2[:, W:], zw], axis=1))
    return acc


def _stats(acc):
    s = jnp.sum(acc, axis=1, keepdims=True)
    ss = jnp.sum(acc * acc, axis=1, keepdims=True)
    return jnp.concatenate([s, ss], axis=1)


def _bn_fold(st, g_ref, b_ref, count, eps):
    mean = st[:, 0:1] * (1.0 / count)
    var = st[:, 1:2] * (1.0 / count) - mean * mean
    scale = g_ref[...] * lax.rsqrt(var + eps)
    shift = b_ref[...] - mean * scale
    return scale, shift


def _conv1_kernel(x_ref, w_ref, y_ref, st_ref, *, W, P, C):
    xb = x_ref[0].astype(jnp.bfloat16)
    acc = _conv_body(xb, w_ref, W, P, C)
    y_ref[0] = acc.astype(jnp.bfloat16)
    st = _stats(acc)

    @pl.when(pl.program_id(0) == 0)
    def _():
        st_ref[...] = st

    @pl.when(pl.program_id(0) != 0)
    def _():
        st_ref[...] += st


def _conv2_kernel(y1_ref, w_ref, st1_ref, g_ref, b_ref, y_ref, st_ref,
                  *, W, P, C, count, eps):
    scale, shift = _bn_fold(st1_ref[...], g_ref, b_ref, count, eps)
    xf = y1_ref[0].astype(jnp.float32) * scale + shift
    xb = jnp.maximum(xf, 0.0).astype(jnp.bfloat16)
    acc = _conv_body(xb, w_ref, W, P, C)
    y_ref[0] = acc.astype(jnp.bfloat16)
    st = _stats(acc)

    @pl.when(pl.program_id(0) == 0)
    def _():
        st_ref[...] = st

    @pl.when(pl.program_id(0) != 0)
    def _():
        st_ref[...] += st


def _tail_kernel(y2_ref, x_ref, st2_ref, g_ref, b_ref,
                 w1_ref, b1_ref, w2_ref, b2_ref, o_ref, *, P, count, eps):
    scale, shift = _bn_fold(st2_ref[...], g_ref, b_ref, count, eps)
    res = y2_ref[0].astype(jnp.float32) * scale + shift
    res = jnp.maximum(res, 0.0)
    sq = jnp.sum(res, axis=1, keepdims=True) * (1.0 / P)
    e = jnp.maximum(
        jnp.dot(w1_ref[...], sq, preferred_element_type=jnp.float32)
        + b1_ref[...], 0.0)
    gate = jax.nn.sigmoid(
        jnp.dot(w2_ref[...], e, preferred_element_type=jnp.float32)
        + b2_ref[...])
    o_ref[0] = jnp.maximum(res * gate + x_ref[0], 0.0)


def kernel(x, conv1_w, bn1_g, bn1_b, conv2_w, bn2_g, bn2_b,
           fc1_w, fc1_b, fc2_w, fc2_b, eps=1e-5):
    N, C, H, W = x.shape
    P = H * W
    R = N * P
    Cr = fc1_w.shape[0]
    f32 = jnp.float32

    xf = x.reshape(N, C, P)
    w1m = jnp.transpose(conv1_w, (0, 2, 3, 1)).reshape(C, 9 * C).astype(jnp.bfloat16)
    w2m = jnp.transpose(conv2_w, (0, 2, 3, 1)).reshape(C, 9 * C).astype(jnp.bfloat16)

    img = lambda shape: pl.BlockSpec(shape, lambda n: (n,) + (0,) * (len(shape) - 1))
    const = lambda shape: pl.BlockSpec(shape, lambda n: (0,) * len(shape))
    cparams = pltpu.CompilerParams(
        dimension_semantics=("arbitrary",), vmem_limit_bytes=_VMEM_LIMIT)
    conv_cost = pl.CostEstimate(flops=2 * R * 9 * C * C, transcendentals=0,
                                bytes_accessed=6 * R * C + 18 * C * C)

    y1, st1 = pl.pallas_call(
        functools.partial(_conv1_kernel, W=W, P=P, C=C),
        grid=(N,),
        in_specs=[img((1, C, P)), const((C, 9 * C))],
        out_specs=[img((1, C, P)), const((C, 2))],
        out_shape=[jax.ShapeDtypeStruct((N, C, P), jnp.bfloat16),
                   jax.ShapeDtypeStruct((C, 2), f32)],
        compiler_params=cparams, cost_estimate=conv_cost,
    )(xf, w1m)

    y2, st2 = pl.pallas_call(
        functools.partial(_conv2_kernel, W=W, P=P, C=C, count=float(R), eps=eps),
        grid=(N,),
        in_specs=[img((1, C, P)), const((C, 9 * C)), const((C, 2)),
                  const((C, 1)), const((C, 1))],
        out_specs=[img((1, C, P)), const((C, 2))],
        out_shape=[jax.ShapeDtypeStruct((N, C, P), jnp.bfloat16),
                   jax.ShapeDtypeStruct((C, 2), f32)],
        compiler_params=cparams, cost_estimate=conv_cost,
    )(y1, w2m, st1, bn1_g.reshape(C, 1).astype(f32),
      bn1_b.reshape(C, 1).astype(f32))

    out = pl.pallas_call(
        functools.partial(_tail_kernel, P=P, count=float(R), eps=eps),
        grid=(N,),
        in_specs=[img((1, C, P)), img((1, C, P)), const((C, 2)),
                  const((C, 1)), const((C, 1)),
                  const((Cr, C)), const((Cr, 1)),
                  const((C, Cr)), const((C, 1))],
        out_specs=img((1, C, P)),
        out_shape=jax.ShapeDtypeStruct((N, C, P), f32),
        compiler_params=cparams,
    )(y2, xf, st2, bn2_g.reshape(C, 1).astype(f32), bn2_b.reshape(C, 1).astype(f32),
      fc1_w.astype(f32), fc1_b.reshape(Cr, 1).astype(f32),
      fc2_w.astype(f32), fc2_b.reshape(C, 1).astype(f32))

    return out.reshape(N, C, H, W)


# speedup vs baseline: 2.0813x; 1.1810x over previous
import functools

import jax
import jax.numpy as jnp
from jax import lax
from jax.experimental import pallas as pl
from jax.experimental.pallas import tpu as pltpu

_VMEM_LIMIT = 48 * 1024 * 1024


def _conv_body(xb, w_ref, W, P, C):
    z1 = jnp.zeros((C, 1), xb.dtype)
    ww = lax.broadcasted_iota(jnp.int32, (1, P), 1) % W
    left = jnp.concatenate([xb[:, 1:], z1], axis=1)
    right = jnp.concatenate([z1, xb[:, :-1]], axis=1)
    left = jnp.where(ww < W - 1, left, jnp.zeros_like(left))
    right = jnp.where(ww >= 1, right, jnp.zeros_like(right))
    x3 = jnp.concatenate([right, xb, left], axis=0)

    b0 = jnp.dot(w_ref[:, 0:3 * C], x3, preferred_element_type=jnp.float32)
    b1 = jnp.dot(w_ref[:, 3 * C:6 * C], x3, preferred_element_type=jnp.float32)
    b2 = jnp.dot(w_ref[:, 6 * C:9 * C], x3, preferred_element_type=jnp.float32)
    zw = jnp.zeros((C, W), jnp.float32)
    acc = (jnp.concatenate([zw, b0[:, :P - W]], axis=1) + b1
           + jnp.concatenate([b2[:, W:], zw], axis=1))
    return acc


def _stats(acc):
    s = jnp.sum(acc, axis=1, keepdims=True)
    ss = jnp.sum(acc * acc, axis=1, keepdims=True)
    return jnp.concatenate([s, ss], axis=1)


def _bn_fold(st, g_ref, b_ref, count, eps):
    mean = st[:, 0:1] * (1.0 / count)
    var = st[:, 1:2] * (1.0 / count) - mean * mean
    scale = g_ref[...] * lax.rsqrt(var + eps)
    shift = b_ref[...] - mean * scale
    return scale, shift


def _accum_st(st_ref, st):
    @pl.when(pl.program_id(0) == 0)
    def _():
        st_ref[...] = st

    @pl.when(pl.program_id(0) != 0)
    def _():
        st_ref[...] += st


def _conv1_kernel(x_ref, w_ref, y_ref, st_ref, *, W, P, C, IMG):
    st = None
    for k in range(IMG):
        xb = x_ref[k].astype(jnp.bfloat16)
        acc = _conv_body(xb, w_ref, W, P, C)
        y_ref[k] = acc.astype(jnp.bfloat16)
        s = _stats(acc)
        st = s if st is None else st + s
    _accum_st(st_ref, st)


def _conv2_kernel(y1_ref, w_ref, st1_ref, g_ref, b_ref, y_ref, st_ref,
                  *, W, P, C, IMG, count, eps):
    scale, shift = _bn_fold(st1_ref[...], g_ref, b_ref, count, eps)
    st = None
    for k in range(IMG):
        xf = y1_ref[k].astype(jnp.float32) * scale + shift
        xb = jnp.maximum(xf, 0.0).astype(jnp.bfloat16)
        acc = _conv_body(xb, w_ref, W, P, C)
        y_ref[k] = acc.astype(jnp.bfloat16)
        s = _stats(acc)
        st = s if st is None else st + s
    _accum_st(st_ref, st)


def _tail_kernel(y2_ref, x_ref, st2_ref, g_ref, b_ref,
                 w1_ref, b1_ref, w2_ref, b2_ref, o_ref, *, P, IMG, count, eps):
    scale, shift = _bn_fold(st2_ref[...], g_ref, b_ref, count, eps)
    for k in range(IMG):
        res = y2_ref[k].astype(jnp.float32) * scale + shift
        res = jnp.maximum(res, 0.0)
        sq = jnp.sum(res, axis=1, keepdims=True) * (1.0 / P)
        e = jnp.maximum(
            jnp.dot(w1_ref[...], sq, preferred_element_type=jnp.float32)
            + b1_ref[...], 0.0)
        gate = jax.nn.sigmoid(
            jnp.dot(w2_ref[...], e, preferred_element_type=jnp.float32)
            + b2_ref[...])
        o_ref[k] = jnp.maximum(res * gate + x_ref[k], 0.0)


def kernel(x, conv1_w, bn1_g, bn1_b, conv2_w, bn2_g, bn2_b,
           fc1_w, fc1_b, fc2_w, fc2_b, eps=1e-5):
    N, C, H, W = x.shape
    P = H * W
    R = N * P
    Cr = fc1_w.shape[0]
    f32 = jnp.float32
    IMG = next(m for m in (8, 4, 2, 1) if N % m == 0)
    G = N // IMG

    xf = x.reshape(N, C, P)
    w1m = jnp.transpose(conv1_w, (0, 2, 3, 1)).reshape(C, 9 * C).astype(jnp.bfloat16)
    w2m = jnp.transpose(conv2_w, (0, 2, 3, 1)).reshape(C, 9 * C).astype(jnp.bfloat16)

    img = lambda shape: pl.BlockSpec(shape, lambda n: (n,) + (0,) * (len(shape) - 1))
    const = lambda shape: pl.BlockSpec(shape, lambda n: (0,) * len(shape))
    cparams = pltpu.CompilerParams(
        dimension_semantics=("arbitrary",), vmem_limit_bytes=_VMEM_LIMIT)
    conv_cost = pl.CostEstimate(flops=2 * R * 9 * C * C, transcendentals=0,
                                bytes_accessed=6 * R * C + 18 * C * C)

    y1, st1 = pl.pallas_call(
        functools.partial(_conv1_kernel, W=W, P=P, C=C, IMG=IMG),
        grid=(G,),
        in_specs=[img((IMG, C, P)), const((C, 9 * C))],
        out_specs=[img((IMG, C, P)), const((C, 2))],
        out_shape=[jax.ShapeDtypeStruct((N, C, P), jnp.bfloat16),
                   jax.ShapeDtypeStruct((C, 2), f32)],
        compiler_params=cparams, cost_estimate=conv_cost,
    )(xf, w1m)

    y2, st2 = pl.pallas_call(
        functools.partial(_conv2_kernel, W=W, P=P, C=C, IMG=IMG,
                          count=float(R), eps=eps),
        grid=(G,),
        in_specs=[img((IMG, C, P)), const((C, 9 * C)), const((C, 2)),
                  const((C, 1)), const((C, 1))],
        out_specs=[img((IMG, C, P)), const((C, 2))],
        out_shape=[jax.ShapeDtypeStruct((N, C, P), jnp.bfloat16),
                   jax.ShapeDtypeStruct((C, 2), f32)],
        compiler_params=cparams, cost_estimate=conv_cost,
    )(y1, w2m, st1, bn1_g.reshape(C, 1).astype(f32),
      bn1_b.reshape(C, 1).astype(f32))

    out = pl.pallas_call(
        functools.partial(_tail_kernel, P=P, IMG=IMG, count=float(R), eps=eps),
        grid=(G,),
        in_specs=[img((IMG, C, P)), img((IMG, C, P)), const((C, 2)),
                  const((C, 1)), const((C, 1)),
                  const((Cr, C)), const((Cr, 1)),
                  const((C, Cr)), const((C, 1))],
        out_specs=img((IMG, C, P)),
        out_shape=jax.ShapeDtypeStruct((N, C, P), f32),
        compiler_params=cparams,
    )(y2, xf, st2, bn2_g.reshape(C, 1).astype(f32), bn2_b.reshape(C, 1).astype(f32),
      fc1_w.astype(f32), fc1_b.reshape(Cr, 1).astype(f32),
      fc2_w.astype(f32), fc2_b.reshape(C, 1).astype(f32))

    return out.reshape(N, C, H, W)


# speedup vs baseline: 2.0856x; 1.0021x over previous
import functools

import jax
import jax.numpy as jnp
from jax import lax
from jax.experimental import pallas as pl
from jax.experimental.pallas import tpu as pltpu

_VMEM_LIMIT = 62 * 1024 * 1024


def _conv_body(xb, wm, W, P, C):
    z1 = jnp.zeros((C, 1), xb.dtype)
    ww = lax.broadcasted_iota(jnp.int32, (1, P), 1) % W
    left = jnp.concatenate([xb[:, 1:], z1], axis=1)
    right = jnp.concatenate([z1, xb[:, :-1]], axis=1)
    left = jnp.where(ww < W - 1, left, jnp.zeros_like(left))
    right = jnp.where(ww >= 1, right, jnp.zeros_like(right))
    x3 = jnp.concatenate([right, xb, left], axis=0)

    b0 = jnp.dot(wm[:, 0:3 * C], x3, preferred_element_type=jnp.float32)
    b1 = jnp.dot(wm[:, 3 * C:6 * C], x3, preferred_element_type=jnp.float32)
    b2 = jnp.dot(wm[:, 6 * C:9 * C], x3, preferred_element_type=jnp.float32)
    zw = jnp.zeros((C, W), jnp.float32)
    return (jnp.concatenate([zw, b0[:, :P - W]], axis=1) + b1
            + jnp.concatenate([b2[:, W:], zw], axis=1))


def _stats(acc):
    s = jnp.sum(acc, axis=1, keepdims=True)
    ss = jnp.sum(acc * acc, axis=1, keepdims=True)
    return jnp.concatenate([s, ss], axis=1)


def _bn_fold(st, g, b, count, eps):
    mean = st[:, 0:1] * (1.0 / count)
    var = st[:, 1:2] * (1.0 / count) - mean * mean
    scale = g * lax.rsqrt(var + eps)
    shift = b - mean * scale
    return scale, shift


def _fused_kernel(x_ref, w_ref, pc_ref, pr_ref, out_ref,
                  y1_s, y2_s, xb_s, st1_s, st2_s, ob_s, sem,
                  *, G, IMG, W, P, C, Cr, count, eps):
    i = pl.program_id(0)

    @pl.when(i < G)
    def _phase1():
        st = None
        for k in range(IMG):
            xb = x_ref[k].astype(jnp.bfloat16)
            xb_s[i * IMG + k] = xb
            acc = _conv_body(xb, w_ref[0:C], W, P, C)
            y1_s[i * IMG + k] = acc.astype(jnp.bfloat16)
            s = _stats(acc)
            st = s if st is None else st + s
        st = st

        @pl.when(i == 0)
        def _():
            st1_s[...] = st

        @pl.when(i != 0)
        def _():
            st1_s[...] += st

    @pl.when((i >= G) & (i < 2 * G))
    def _phase2():
        j = i - G
        scale, shift = _bn_fold(st1_s[...], pc_ref[:, 0:1], pc_ref[:, 1:2],
                                count, eps)
        st = None
        for k in range(IMG):
            xf = y1_s[j * IMG + k].astype(jnp.float32) * scale + shift
            xb = jnp.maximum(xf, 0.0).astype(jnp.bfloat16)
            acc = _conv_body(xb, w_ref[C:2 * C], W, P, C)
            y2_s[j * IMG + k] = acc.astype(jnp.bfloat16)
            s = _stats(acc)
            st = s if st is None else st + s
        st = st

        @pl.when(j == 0)
        def _():
            st2_s[...] = st

        @pl.when(j != 0)
        def _():
            st2_s[...] += st

    @pl.when(i >= 2 * G)
    def _phase3():
        j = i - 2 * G
        buf = lax.rem(j, 2)
        scale, shift = _bn_fold(st2_s[...], pc_ref[:, 2:3], pc_ref[:, 3:4],
                                count, eps)
        fc2_w = pc_ref[:, 4:4 + Cr]
        fc2_b = pc_ref[:, 4 + Cr:5 + Cr]
        fc1_w = pr_ref[:, 0:C]
        fc1_b = pr_ref[:, C:C + 1]

        @pl.when(j >= 2)
        def _():
            pltpu.make_async_copy(
                ob_s.at[buf], out_ref.at[pl.ds((j - 2) * IMG, IMG)],
                sem.at[buf]).wait()

        for k in range(IMG):
            res = y2_s[j * IMG + k].astype(jnp.float32) * scale + shift
            res = jnp.maximum(res, 0.0)
            sq = jnp.sum(res, axis=1, keepdims=True) * (1.0 / P)
            e = jnp.maximum(
                jnp.dot(fc1_w, sq, preferred_element_type=jnp.float32)
                + fc1_b, 0.0)
            gate = jax.nn.sigmoid(
                jnp.dot(fc2_w, e, preferred_element_type=jnp.float32)
                + fc2_b)
            ob_s[buf, k] = jnp.maximum(
                res * gate + xb_s[j * IMG + k].astype(jnp.float32), 0.0)

        pltpu.make_async_copy(
            ob_s.at[buf], out_ref.at[pl.ds(j * IMG, IMG)], sem.at[buf]).start()

        @pl.when(j == G - 1)
        def _():
            if G > 1:
                pltpu.make_async_copy(
                    ob_s.at[1 - buf], out_ref.at[pl.ds((j - 1) * IMG, IMG)],
                    sem.at[1 - buf]).wait()
            pltpu.make_async_copy(
                ob_s.at[buf], out_ref.at[pl.ds(j * IMG, IMG)],
                sem.at[buf]).wait()


def kernel(x, conv1_w, bn1_g, bn1_b, conv2_w, bn2_g, bn2_b,
           fc1_w, fc1_b, fc2_w, fc2_b, eps=1e-5):
    N, C, H, W = x.shape
    P = H * W
    R = N * P
    Cr = fc1_w.shape[0]
    f32 = jnp.float32
    IMG = 2 if N % 2 == 0 else 1
    G = N // IMG

    xf = x.reshape(N, C, P)
    wm = jnp.concatenate(
        [jnp.transpose(conv1_w, (0, 2, 3, 1)).reshape(C, 9 * C),
         jnp.transpose(conv2_w, (0, 2, 3, 1)).reshape(C, 9 * C)],
        axis=0).astype(jnp.bfloat16)
    pc = jnp.concatenate(
        [bn1_g.reshape(C, 1), bn1_b.reshape(C, 1),
         bn2_g.reshape(C, 1), bn2_b.reshape(C, 1),
         fc2_w.reshape(C, Cr), fc2_b.reshape(C, 1)], axis=1).astype(f32)
    pr = jnp.concatenate(
        [fc1_w.reshape(Cr, C), fc1_b.reshape(Cr, 1)], axis=1).astype(f32)

    out = pl.pallas_call(
        functools.partial(_fused_kernel, G=G, IMG=IMG, W=W, P=P, C=C, Cr=Cr,
                          count=float(R), eps=eps),
        grid=(3 * G,),
        in_specs=[
            pl.BlockSpec((IMG, C, P), lambda n: (jnp.minimum(n, G - 1), 0, 0)),
            pl.BlockSpec((2 * C, 9 * C), lambda n: (0, 0)),
            pl.BlockSpec((C, 5 + Cr), lambda n: (0, 0)),
            pl.BlockSpec((Cr, C + 1), lambda n: (0, 0)),
        ],
        out_specs=pl.BlockSpec(memory_space=pl.ANY),
        out_shape=jax.ShapeDtypeStruct((N, C, P), f32),
        scratch_shapes=[
            pltpu.VMEM((N, C, P), jnp.bfloat16),
            pltpu.VMEM((N, C, P), jnp.bfloat16),
            pltpu.VMEM((N, C, P), jnp.bfloat16),
            pltpu.VMEM((C, 2), f32),
            pltpu.VMEM((C, 2), f32),
            pltpu.VMEM((2, IMG, C, P), f32),
            pltpu.SemaphoreType.DMA((2,)),
        ],
        compiler_params=pltpu.CompilerParams(
            dimension_semantics=("arbitrary",), vmem_limit_bytes=_VMEM_LIMIT),
        cost_estimate=pl.CostEstimate(
            flops=4 * R * 9 * C * C, transcendentals=N * C,
            bytes_accessed=8 * R * C),
    )(xf, wm, pc, pr)

    return out.reshape(N, C, H, W)


# speedup vs baseline: 2.1354x; 1.0238x over previous
import functools

import jax
import jax.numpy as jnp
from jax import lax
from jax.experimental import pallas as pl
from jax.experimental.pallas import tpu as pltpu

_VMEM_LIMIT = 62 * 1024 * 1024


def _conv_body(xb, wm, W, P, C):
    z1 = jnp.zeros((C, 1), xb.dtype)
    ww = lax.broadcasted_iota(jnp.int32, (1, P), 1) % W
    left = jnp.concatenate([xb[:, 1:], z1], axis=1)
    right = jnp.concatenate([z1, xb[:, :-1]], axis=1)
    left = jnp.where(ww < W - 1, left, jnp.zeros_like(left))
    right = jnp.where(ww >= 1, right, jnp.zeros_like(right))
    x3 = jnp.concatenate([right, xb, left], axis=0)

    b0 = jnp.dot(wm[:, 0:3 * C], x3, preferred_element_type=jnp.float32)
    b1 = jnp.dot(wm[:, 3 * C:6 * C], x3, preferred_element_type=jnp.float32)
    b2 = jnp.dot(wm[:, 6 * C:9 * C], x3, preferred_element_type=jnp.float32)
    zw = jnp.zeros((C, W), jnp.float32)
    return (jnp.concatenate([zw, b0[:, :P - W]], axis=1) + b1
            + jnp.concatenate([b2[:, W:], zw], axis=1))


def _stats(acc):
    s = jnp.sum(acc, axis=1, keepdims=True)
    ss = jnp.sum(acc * acc, axis=1, keepdims=True)
    return jnp.concatenate([s, ss], axis=1)


def _bn_fold(st, g, b, count, eps):
    mean = st[:, 0:1] * (1.0 / count)
    var = st[:, 1:2] * (1.0 / count) - mean * mean
    scale = g * lax.rsqrt(var + eps)
    shift = b - mean * scale
    return scale, shift


def _fused_kernel(x_ref, w_ref, pc_ref, pr_ref, out_ref,
                  y1_s, y2_s, xb_s, st1_s, st2_s, ob_s, sem,
                  *, G1, IMG1, G3, IMG3, W, P, C, Cr, count, eps):
    i = pl.program_id(0)

    @pl.when(i < G1)
    def _phase1():
        st = None
        for k in range(IMG1):
            xb = x_ref[k].astype(jnp.bfloat16)
            xb_s[i * IMG1 + k] = xb
            acc = _conv_body(xb, w_ref[0:C], W, P, C)
            y1_s[i * IMG1 + k] = acc.astype(jnp.bfloat16)
            s = _stats(acc)
            st = s if st is None else st + s
        st = st

        @pl.when(i == 0)
        def _():
            st1_s[...] = st

        @pl.when(i != 0)
        def _():
            st1_s[...] += st

    @pl.when((i >= G1) & (i < 2 * G1))
    def _phase2():
        j = i - G1
        scale, shift = _bn_fold(st1_s[...], pc_ref[:, 0:1], pc_ref[:, 1:2],
                                count, eps)
        st = None
        for k in range(IMG1):
            xf = y1_s[j * IMG1 + k].astype(jnp.float32) * scale + shift
            xb = jnp.maximum(xf, 0.0).astype(jnp.bfloat16)
            acc = _conv_body(xb, w_ref[C:2 * C], W, P, C)
            y2_s[j * IMG1 + k] = acc.astype(jnp.bfloat16)
            s = _stats(acc)
            st = s if st is None else st + s
        st = st

        @pl.when(j == 0)
        def _():
            st2_s[...] = st

        @pl.when(j != 0)
        def _():
            st2_s[...] += st

    @pl.when(i >= 2 * G1)
    def _phase3():
        j = i - 2 * G1
        buf = lax.rem(j, 2)
        scale, shift = _bn_fold(st2_s[...], pc_ref[:, 2:3], pc_ref[:, 3:4],
                                count, eps)
        fc2_w = pc_ref[:, 4:4 + Cr]
        fc2_b = pc_ref[:, 4 + Cr:5 + Cr]
        fc1_w = pr_ref[:, 0:C]
        fc1_b = pr_ref[:, C:C + 1]

        @pl.when(j >= 2)
        def _():
            pltpu.make_async_copy(
                ob_s.at[buf], out_ref.at[pl.ds((j - 2) * IMG3, IMG3)],
                sem.at[buf]).wait()

        for k in range(IMG3):
            res = y2_s[j * IMG3 + k].astype(jnp.float32) * scale + shift
            res = jnp.maximum(res, 0.0)
            sq = jnp.dot(res, jnp.ones((P, 1), jnp.float32),
                         preferred_element_type=jnp.float32) * (1.0 / P)
            e = jnp.maximum(
                jnp.dot(fc1_w, sq, preferred_element_type=jnp.float32)
                + fc1_b, 0.0)
            gate = jax.nn.sigmoid(
                jnp.dot(fc2_w, e, preferred_element_type=jnp.float32)
                + fc2_b)
            ob_s[buf, k] = jnp.maximum(
                res * gate + xb_s[j * IMG3 + k].astype(jnp.float32), 0.0)

        pltpu.make_async_copy(
            ob_s.at[buf], out_ref.at[pl.ds(j * IMG3, IMG3)], sem.at[buf]).start()

        @pl.when(j == G3 - 1)
        def _():
            if G3 > 1:
                pltpu.make_async_copy(
                    ob_s.at[1 - buf], out_ref.at[pl.ds((j - 1) * IMG3, IMG3)],
                    sem.at[1 - buf]).wait()
            pltpu.make_async_copy(
                ob_s.at[buf], out_ref.at[pl.ds(j * IMG3, IMG3)],
                sem.at[buf]).wait()


def kernel(x, conv1_w, bn1_g, bn1_b, conv2_w, bn2_g, bn2_b,
           fc1_w, fc1_b, fc2_w, fc2_b, eps=1e-5):
    N, C, H, W = x.shape
    P = H * W
    R = N * P
    Cr = fc1_w.shape[0]
    f32 = jnp.float32
    IMG1 = next(m for m in (4, 2, 1) if N % m == 0)
    IMG3 = 2 if N % 2 == 0 else 1
    G1 = N // IMG1
    G3 = N // IMG3

    xf = x.reshape(N, C, P)
    wm = jnp.concatenate(
        [jnp.transpose(conv1_w, (0, 2, 3, 1)).reshape(C, 9 * C),
         jnp.transpose(conv2_w, (0, 2, 3, 1)).reshape(C, 9 * C)],
        axis=0).astype(jnp.bfloat16)
    pc = jnp.concatenate(
        [bn1_g.reshape(C, 1), bn1_b.reshape(C, 1),
         bn2_g.reshape(C, 1), bn2_b.reshape(C, 1),
         fc2_w.reshape(C, Cr), fc2_b.reshape(C, 1)], axis=1).astype(f32)
    pr = jnp.concatenate(
        [fc1_w.reshape(Cr, C), fc1_b.reshape(Cr, 1)], axis=1).astype(f32)

    out = pl.pallas_call(
        functools.partial(_fused_kernel, G1=G1, IMG1=IMG1, G3=G3, IMG3=IMG3,
                          W=W, P=P, C=C, Cr=Cr, count=float(R), eps=eps),
        grid=(2 * G1 + G3,),
        in_specs=[
            pl.BlockSpec((IMG1, C, P), lambda n: (jnp.minimum(n, G1 - 1), 0, 0)),
            pl.BlockSpec((2 * C, 9 * C), lambda n: (0, 0)),
            pl.BlockSpec((C, 5 + Cr), lambda n: (0, 0)),
            pl.BlockSpec((Cr, C + 1), lambda n: (0, 0)),
        ],
        out_specs=pl.BlockSpec(memory_space=pl.ANY),
        out_shape=jax.ShapeDtypeStruct((N, C, P), f32),
        scratch_shapes=[
            pltpu.VMEM((N, C, P), jnp.bfloat16),
            pltpu.VMEM((N, C, P), jnp.bfloat16),
            pltpu.VMEM((N, C, P), jnp.bfloat16),
            pltpu.VMEM((C, 2), f32),
            pltpu.VMEM((C, 2), f32),
            pltpu.VMEM((2, IMG3, C, P), f32),
            pltpu.SemaphoreType.DMA((2,)),
        ],
        compiler_params=pltpu.CompilerParams(
            dimension_semantics=("arbitrary",), vmem_limit_bytes=_VMEM_LIMIT),
        cost_estimate=pl.CostEstimate(
            flops=4 * R * 9 * C * C, transcendentals=N * C,
            bytes_accessed=8 * R * C),
    )(xf, wm, pc, pr)

    return out.reshape(N, C, H, W)
